# baseline (speedup 1.0000x reference)
import functools

import jax
import jax.numpy as jnp
from jax import lax
from jax.experimental import pallas as pl
from jax.experimental.pallas import tpu as pltpu
from jax.experimental.pallas import tpu_sc as plsc

_L = 16


def _transposed_lookup(hair, eyes, htab_t, etab_t):
    B = hair.shape[0]
    D, V = htab_t.shape
    half = V // 2
    n_vec = B // _L
    mesh = plsc.VectorSubcoreMesh(core_axis_name="c", subcore_axis_name="s")
    info = plsc.get_sparse_core_info()

    @functools.partial(
        pl.kernel,
        mesh=mesh,
        compiler_params=pltpu.CompilerParams(
            use_tc_tiling_on_sc=False, needs_layout_passes=False),
        out_type=(
            jax.ShapeDtypeStruct((D, B), jnp.float32),
            jax.ShapeDtypeStruct((D, B), jnp.float32),
        ),
        scratch_types=[
            pltpu.VMEM((B,), jnp.int32),
            pltpu.VMEM((half,), jnp.float32),
            pltpu.VMEM((B,), jnp.float32),
        ],
    )
    def body(hair_hbm, eyes_hbm, htab_hbm, etab_hbm, hout_hbm, eout_hbm,
             idx_v, col_v, out_v):
        wid = lax.axis_index("s") * info.num_cores + lax.axis_index("c")
        lane = lax.iota(jnp.int32, _L)

        for idx_hbm, tab_hbm, out_hbm in (
            (hair_hbm, htab_hbm, hout_hbm),
            (eyes_hbm, etab_hbm, eout_hbm),
        ):
            pltpu.sync_copy(idx_hbm, idx_v)
            for p in range(2):
                lo = p * half
                pltpu.sync_copy(tab_hbm.at[wid, pl.ds(lo, half)], col_v)

                def gather_chunk(i, _, lo=lo):
                    base = i * _L
                    idxv = idx_v[pl.ds(base, _L)]
                    rel = idxv - lo
                    m = (idxv >= lo) & (idxv < lo + half)
                    vals = plsc.load_gather(col_v, [rel], mask=m)
                    plsc.store_scatter(out_v, [base + lane], vals, mask=m)
                    return 0

                lax.fori_loop(0, n_vec, gather_chunk, 0)
            pltpu.sync_copy(out_v, out_hbm.at[wid])

    return body(hair, eyes, htab_t, etab_t)


@jax.jit
def _lookup(hair, eyes, hair_table, eye_table):
    ht, et = _transposed_lookup(hair, eyes, hair_table.T, eye_table.T)
    return ht.T, et.T


def kernel(hair, eyes, hair_table, eye_table):
    return _lookup(hair, eyes, hair_table, eye_table)


# speedup vs baseline: 1.2581x; 1.2581x over previous
import functools

import jax
import jax.numpy as jnp
from jax import lax
from jax.experimental import pallas as pl
from jax.experimental.pallas import tpu as pltpu
from jax.experimental.pallas import tpu_sc as plsc

_L = 16


def _transposed_lookup(hair, eyes, htab_t, etab_t):
    B = hair.shape[0]
    D, V = htab_t.shape
    n_vec = B // _L
    mesh = plsc.VectorSubcoreMesh(core_axis_name="c", subcore_axis_name="s")
    info = plsc.get_sparse_core_info()

    @functools.partial(
        pl.kernel,
        mesh=mesh,
        compiler_params=pltpu.CompilerParams(
            use_tc_tiling_on_sc=False, needs_layout_passes=False),
        out_type=(
            jax.ShapeDtypeStruct((D, B), jnp.int32),
            jax.ShapeDtypeStruct((D, B), jnp.int32),
        ),
        scratch_types=[
            pltpu.VMEM((B,), jnp.int32),
            pltpu.VMEM((V,), jnp.float32),
        ],
    )
    def body(hair_hbm, eyes_hbm, htab_hbm, etab_hbm, hout_hbm, eout_hbm,
             iob_v, col_v):
        wid = lax.axis_index("s") * info.num_cores + lax.axis_index("c")

        for idx_hbm, tab_hbm, out_hbm in (
            (hair_hbm, htab_hbm, hout_hbm),
            (eyes_hbm, etab_hbm, eout_hbm),
        ):
            pltpu.sync_copy(idx_hbm, iob_v)
            pltpu.sync_copy(tab_hbm.at[wid], col_v)

            def gather_chunk(i, _):
                sl = pl.ds(i * _L, _L)
                vals = plsc.load_gather(col_v, [iob_v[sl]])
                iob_v[sl] = plsc.bitcast(vals, jnp.int32)
                return 0

            lax.fori_loop(0, n_vec, gather_chunk, 0)
            pltpu.sync_copy(iob_v, out_hbm.at[wid])

    return body(hair, eyes, htab_t, etab_t)


@jax.jit
def _lookup(hair, eyes, hair_table, eye_table):
    ht, et = _transposed_lookup(hair, eyes, hair_table.T, eye_table.T)
    ht = lax.bitcast_convert_type(ht, jnp.float32)
    et = lax.bitcast_convert_type(et, jnp.float32)
    return ht.T, et.T


def kernel(hair, eyes, hair_table, eye_table):
    return _lookup(hair, eyes, hair_table, eye_table)


# speedup vs baseline: 1.6857x; 1.3399x over previous
import functools

import jax
import jax.numpy as jnp
from jax import lax
from jax.experimental import pallas as pl
from jax.experimental.pallas import tpu as pltpu
from jax.experimental.pallas import tpu_sc as plsc

_L = 16


def _one_lookup(idx, tab_t):
    B = idx.shape[0]
    D, V = tab_t.shape
    n_vec = B // _L
    mesh = plsc.VectorSubcoreMesh(core_axis_name="c", subcore_axis_name="s")
    info = plsc.get_sparse_core_info()

    @functools.partial(
        pl.kernel,
        mesh=mesh,
        compiler_params=pltpu.CompilerParams(
            use_tc_tiling_on_sc=False, needs_layout_passes=False),
        out_type=jax.ShapeDtypeStruct((D, B), jnp.int32),
        scratch_types=[
            pltpu.VMEM((B,), jnp.int32),
            pltpu.VMEM((V,), jnp.float32),
            pltpu.SemaphoreType.DMA,
            pltpu.SemaphoreType.DMA,
        ],
    )
    def body(idx_hbm, tab_hbm, out_hbm, iob_v, col_v, sem_i, sem_c):
        wid = lax.axis_index("s") * info.num_cores + lax.axis_index("c")
        ci = pltpu.async_copy(idx_hbm, iob_v, sem_i)
        cc = pltpu.async_copy(tab_hbm.at[wid], col_v, sem_c)
        ci.wait()
        cc.wait()

        @plsc.parallel_loop(0, n_vec, unroll=8)
        def gather_chunk(i):
            sl = pl.ds(i * _L, _L)
            iob_v[sl] = plsc.bitcast(
                plsc.load_gather(col_v, [iob_v[sl]]), jnp.int32)

        pltpu.sync_copy(iob_v, out_hbm.at[wid])

    return body(idx, tab_t)


@jax.jit
def _lookup(hair, eyes, hair_table, eye_table):
    ht = _one_lookup(hair, hair_table.T)
    et = _one_lookup(eyes, eye_table.T)
    ht = lax.bitcast_convert_type(ht, jnp.float32)
    et = lax.bitcast_convert_type(et, jnp.float32)
    return ht.T, et.T


def kernel(hair, eyes, hair_table, eye_table):
    return _lookup(hair, eyes, hair_table, eye_table)
